# baseline (speedup 1.0000x reference)
import functools
import numpy as np

import jax
import jax.numpy as jnp
from jax import lax
from jax.experimental import pallas as pl
from jax.experimental.pallas import tpu as pltpu
from jax.experimental.pallas import tpu_sc as plsc

B = 128; T = 4; D = 128; QD = 128; VD = 64; DM = 64; M = 2048
T0 = 256; T1 = 256; OUT = 256; RK = 8; WK = 8

BB = 8

_RSQRT_QD = 1.0 / float(np.sqrt(QD))
_RSQRT_DM = 1.0 / float(np.sqrt(DM))


def _rbf16(x):
    u = jax.lax.bitcast_convert_type(x, jnp.uint32)
    lsb = jax.lax.shift_right_logical(u, jnp.uint32(16)) & jnp.uint32(1)
    r = (u + jnp.uint32(0x7FFF) + lsb) & jnp.uint32(0xFFFF0000)
    return jax.lax.bitcast_convert_type(r, jnp.float32)


def _mm(a, b):
    return jax.lax.dot_general(_rbf16(a), _rbf16(b), (((1,), (0,)), ((), ())),
                               preferred_element_type=jnp.float32)


def _bmv(mat, vec):
    return jax.lax.dot_general(_rbf16(mat), _rbf16(vec),
                               (((2,), (1,)), ((0,), (0,))),
                               preferred_element_type=jnp.float32)


def _bmv_pre(mat, vec):
    return jax.lax.dot_general(mat, _rbf16(vec),
                               (((2,), (1,)), ((0,), (0,))),
                               preferred_element_type=jnp.float32)


def _bvm(vec, mat):
    return jax.lax.dot_general(_rbf16(vec), _rbf16(mat),
                               (((1,), (1,)), ((0,), (0,))),
                               preferred_element_type=jnp.float32)


def _topk(scores, iota_m, k):
    vals, idxs = [], []
    s = scores
    neg = jnp.float32(-jnp.inf)
    for _ in range(k):
        mv = jnp.max(s, axis=-1, keepdims=True)
        im = jnp.min(jnp.where(s == mv, iota_m, M), axis=-1, keepdims=True)
        vals.append(mv)
        idxs.append(im)
        s = jnp.where(iota_m == im, neg, s)
    return vals, idxs


def _scalar_i(x):
    return jnp.max(x)


def _body(e_ref, mem_ref, ema_ref, lw_ref,
          wq, wk, wv, wr, wt0, bt0, wt1, bt1, ww, bw, wg, bg, wh, bh,
          gam, bet, out_ref, mem_bf):
    mem_bf[...] = _rbf16(mem_ref[...])
    ema = ema_ref[...]
    lw = lw_ref[...]
    s0 = jnp.zeros((BB, T0), jnp.float32)
    s1 = jnp.zeros((BB, T1), jnp.float32)
    es = [e_ref[:, t, :] for t in range(T)]
    iota_m = lax.broadcasted_iota(jnp.int32, (BB, M), 1)

    for t in range(T):
        e_t = es[t]
        if t == 0:
            local_read = jnp.zeros((BB, VD), jnp.float32)
        else:
            q = _mm(e_t, wq[...])
            kstk = jnp.concatenate(
                [_mm(es[j], wk[...])[:, None, :] for j in range(t)], axis=1)
            sc = _bmv(kstk, q) / np.sqrt(QD)
            mx = jnp.max(sc, axis=-1, keepdims=True)
            ex = jnp.exp(sc - mx)
            attn = ex / jnp.sum(ex, axis=-1, keepdims=True)
            vstk = jnp.concatenate(
                [_mm(es[j], wv[...])[:, None, :] for j in range(t)], axis=1)
            local_read = _bvm(attn, vstk)
        u = jnp.concatenate([e_t, s0, s1], axis=-1)
        qm = _mm(u, wr[...])
        scores = _bmv_pre(mem_bf[...], qm) / np.sqrt(DM)
        topv, topi = _topk(scores, iota_m, RK)
        tv = jnp.concatenate(topv, axis=1)
        ex = jnp.exp(tv - tv[:, 0:1])
        alpha = ex / jnp.sum(ex, axis=-1, keepdims=True)
        rrows = []
        for b in range(BB):
            rk = [mem_ref[b, pl.ds(_scalar_i(topi[k][b:b + 1, :]), 1), :]
                  for k in range(RK)]
            rrows.append(jnp.concatenate(rk, axis=0)[None])
        read_vecs = jnp.concatenate(rrows, axis=0)
        r_t = _bvm(alpha, read_vecs)
        ema = ema * 0.99
        for k in range(RK):
            ema = ema + jnp.where(iota_m == topi[k], alpha[:, k:k + 1], 0.0)
        t0_in = jnp.concatenate([e_t, r_t + local_read, s0], axis=-1)
        s0n = jnp.tanh(_mm(t0_in, wt0[...]) + bt0[...])
        cand1 = jnp.tanh(_mm(jnp.concatenate([s0n, s1], axis=-1), wt1[...])
                         + bt1[...])
        s1n = cand1 if t % 2 == 0 else s1
        wscore = 1.0 / (1.0 + ema) + 0.01 * (float(t) - lw)
        _, wtopi = _topk(wscore, iota_m, WK)
        win = jnp.concatenate([s0n, s1n, r_t], axis=-1)
        w_t = jnp.tanh(_mm(win, ww[...]) + bw[...])
        g_t = jax.nn.sigmoid(_mm(win, wg[...]) + bg[...])
        for b in range(BB):
            g_s = jnp.max(g_t[b:b + 1, :])
            wrow = w_t[b:b + 1, :]
            for k in range(WK):
                idx = _scalar_i(wtopi[k][b:b + 1, :])
                old = mem_ref[b, pl.ds(idx, 1), :]
                new = (1.0 - g_s) * old + g_s * wrow
                mem_ref[b, pl.ds(idx, 1), :] = new
                mem_bf[b, pl.ds(idx, 1), :] = _rbf16(new)
        for k in range(WK):
            lw = jnp.where(iota_m == wtopi[k], float(t), lw)
        cat = jnp.concatenate([s0n, s1n], axis=-1)
        mu = jnp.mean(cat, axis=-1, keepdims=True)
        var = jnp.mean((cat - mu) * (cat - mu), axis=-1, keepdims=True)
        normed = (cat - mu) / jnp.sqrt(var + 1e-5) * gam[...] + bet[...]
        out_ref[:, t, :] = _mm(normed, wh[...]) + bh[...]
        s0, s1 = s0n, s1n


def _sc_embed_gather(emb, ids_flat):
    info = plsc.get_sparse_core_info()
    nw = info.num_cores * info.num_subcores
    bt = B * T
    b_per_w = bt // nw
    mesh = plsc.VectorSubcoreMesh(core_axis_name="c", subcore_axis_name="s")

    @functools.partial(
        pl.kernel, mesh=mesh,
        out_type=jax.ShapeDtypeStruct((bt, D), jnp.float32),
        scratch_types=[
            pltpu.VMEM((b_per_w,), jnp.int32),
            pltpu.VMEM((b_per_w, D), jnp.float32),
            pltpu.SemaphoreType.DMA,
        ],
    )
    def gather(table_hbm, idx_hbm, out_hbm, idx_v, rows_v, sem):
        wid = lax.axis_index("s") * info.num_cores + lax.axis_index("c")
        base = wid * b_per_w
        pltpu.sync_copy(idx_hbm.at[pl.ds(base, b_per_w)], idx_v)
        pltpu.async_copy(table_hbm.at[idx_v], rows_v, sem).wait()
        pltpu.sync_copy(rows_v, out_hbm.at[pl.ds(base, b_per_w)])

    return gather(emb, ids_flat)


def kernel(input_ids, emb, Wq, Wk, Wv, Wr, Wt0, bt0, Wt1, bt1, Ww, bw,
           Wg, bg, Whead, bhead, gamma, beta, memory0, local_buffer0,
           read_ema0, last_write0):
    del local_buffer0
    ids_flat = input_ids.reshape(-1).astype(jnp.int32)
    e_all = _sc_embed_gather(emb, ids_flat).reshape(B, T, D)

    grid = (B // BB,)
    w_spec2 = lambda a: pl.BlockSpec(a.shape, lambda i: (0, 0))

    def row1(x):
        return x.reshape(1, -1)

    out = pl.pallas_call(
        _body,
        grid=grid,
        in_specs=[
            pl.BlockSpec((BB, T, D), lambda i: (i, 0, 0)),
            pl.BlockSpec((BB, M, DM), lambda i: (i, 0, 0)),
            pl.BlockSpec((BB, M), lambda i: (i, 0)),
            pl.BlockSpec((BB, M), lambda i: (i, 0)),
            w_spec2(Wq), w_spec2(Wk), w_spec2(Wv), w_spec2(Wr),
            w_spec2(Wt0), w_spec2(bt0.reshape(1, -1)),
            w_spec2(Wt1), w_spec2(bt1.reshape(1, -1)),
            w_spec2(Ww), w_spec2(bw.reshape(1, -1)),
            w_spec2(Wg), w_spec2(bg.reshape(1, -1)),
            w_spec2(Whead), w_spec2(bhead.reshape(1, -1)),
            w_spec2(gamma.reshape(1, -1)), w_spec2(beta.reshape(1, -1)),
        ],
        out_specs=pl.BlockSpec((BB, T, OUT), lambda i: (i, 0, 0)),
        out_shape=jax.ShapeDtypeStruct((B, T, OUT), jnp.float32),
        scratch_shapes=[pltpu.VMEM((BB, M, DM), jnp.float32)],
        compiler_params=pltpu.CompilerParams(
            dimension_semantics=("arbitrary",),
        ),
    )(e_all, memory0, read_ema0, last_write0,
      Wq, Wk, Wv, Wr, Wt0, row1(bt0), Wt1, row1(bt1), Ww, row1(bw),
      Wg, row1(bg), Whead, row1(bhead), row1(gamma), row1(beta))
    return out


# speedup vs baseline: 1.2242x; 1.2242x over previous
import functools
import numpy as np

import jax
import jax.numpy as jnp
from jax import lax
from jax.experimental import pallas as pl
from jax.experimental.pallas import tpu as pltpu
from jax.experimental.pallas import tpu_sc as plsc

B = 128; T = 4; D = 128; QD = 128; VD = 64; DM = 64; M = 2048
T0 = 256; T1 = 256; OUT = 256; RK = 8; WK = 8

BB = 8
MC = 512

_RSQRT_QD = 1.0 / float(np.sqrt(QD))
_RSQRT_DM = 1.0 / float(np.sqrt(DM))


def _rbf16(x):
    u = jax.lax.bitcast_convert_type(x, jnp.uint32)
    lsb = jax.lax.shift_right_logical(u, jnp.uint32(16)) & jnp.uint32(1)
    r = (u + jnp.uint32(0x7FFF) + lsb) & jnp.uint32(0xFFFF0000)
    return jax.lax.bitcast_convert_type(r, jnp.float32)


def _mm(a, b):
    return jax.lax.dot_general(_rbf16(a), _rbf16(b), (((1,), (0,)), ((), ())),
                               preferred_element_type=jnp.float32)


def _bmv(mat, vec):
    return jax.lax.dot_general(_rbf16(mat), _rbf16(vec),
                               (((2,), (1,)), ((0,), (0,))),
                               preferred_element_type=jnp.float32)


def _bmv_pre(mat, vec):
    return jax.lax.dot_general(mat, _rbf16(vec),
                               (((2,), (1,)), ((0,), (0,))),
                               preferred_element_type=jnp.float32)


def _bvm(vec, mat):
    return jax.lax.dot_general(_rbf16(vec), _rbf16(mat),
                               (((1,), (1,)), ((0,), (0,))),
                               preferred_element_type=jnp.float32)


def _topk(scores, iota_m, k):
    vals, idxs = [], []
    s = scores
    neg = jnp.float32(-jnp.inf)
    for _ in range(k):
        mv = jnp.max(s, axis=-1, keepdims=True)
        im = jnp.min(jnp.where(s == mv, iota_m, M), axis=-1, keepdims=True)
        vals.append(mv)
        idxs.append(im)
        s = jnp.where(iota_m == im, neg, s)
    return vals, idxs


def _scalar_i(x):
    return jnp.max(x)


def _body(e_ref, mem_ref, ema_ref, lw_ref,
          wq, wk, wv, wr, wt0, bt0, wt1, bt1, ww, bw, wg, bg, wh, bh,
          gam, bet, out_ref, mem_bf):
    for mc in range(0, M, MC):
        mem_bf[:, mc:mc + MC, :] = _rbf16(mem_ref[:, mc:mc + MC, :])
    ema = ema_ref[...]
    lw = lw_ref[...]
    s0 = jnp.zeros((BB, T0), jnp.float32)
    s1 = jnp.zeros((BB, T1), jnp.float32)
    es = [e_ref[:, t, :] for t in range(T)]
    iota_m = lax.broadcasted_iota(jnp.int32, (BB, M), 1)

    for t in range(T):
        e_t = es[t]
        if t == 0:
            local_read = jnp.zeros((BB, VD), jnp.float32)
        else:
            q = _mm(e_t, wq[...])
            kstk = jnp.concatenate(
                [_mm(es[j], wk[...])[:, None, :] for j in range(t)], axis=1)
            sc = _bmv(kstk, q) / np.sqrt(QD)
            mx = jnp.max(sc, axis=-1, keepdims=True)
            ex = jnp.exp(sc - mx)
            attn = ex / jnp.sum(ex, axis=-1, keepdims=True)
            vstk = jnp.concatenate(
                [_mm(es[j], wv[...])[:, None, :] for j in range(t)], axis=1)
            local_read = _bvm(attn, vstk)
        u = jnp.concatenate([e_t, s0, s1], axis=-1)
        qm = _mm(u, wr[...])
        scores = jnp.concatenate(
            [_bmv_pre(mem_bf[:, mc:mc + MC, :], qm) for mc in range(0, M, MC)],
            axis=-1) / np.sqrt(DM)
        topv, topi = _topk(scores, iota_m, RK)
        tv = jnp.concatenate(topv, axis=1)
        ex = jnp.exp(tv - tv[:, 0:1])
        alpha = ex / jnp.sum(ex, axis=-1, keepdims=True)
        rrows = []
        for b in range(BB):
            rk = [mem_ref[b, pl.ds(_scalar_i(topi[k][b:b + 1, :]), 1), :]
                  for k in range(RK)]
            rrows.append(jnp.concatenate(rk, axis=0)[None])
        read_vecs = jnp.concatenate(rrows, axis=0)
        r_t = _bvm(alpha, read_vecs)
        ema = ema * 0.99
        for k in range(RK):
            ema = ema + jnp.where(iota_m == topi[k], alpha[:, k:k + 1], 0.0)
        t0_in = jnp.concatenate([e_t, r_t + local_read, s0], axis=-1)
        s0n = jnp.tanh(_mm(t0_in, wt0[...]) + bt0[...])
        cand1 = jnp.tanh(_mm(jnp.concatenate([s0n, s1], axis=-1), wt1[...])
                         + bt1[...])
        s1n = cand1 if t % 2 == 0 else s1
        wscore = 1.0 / (1.0 + ema) + 0.01 * (float(t) - lw)
        _, wtopi = _topk(wscore, iota_m, WK)
        win = jnp.concatenate([s0n, s1n, r_t], axis=-1)
        w_t = jnp.tanh(_mm(win, ww[...]) + bw[...])
        g_t = jax.nn.sigmoid(_mm(win, wg[...]) + bg[...])
        for b in range(BB):
            g_s = jnp.max(g_t[b:b + 1, :])
            wrow = w_t[b:b + 1, :]
            for k in range(WK):
                idx = _scalar_i(wtopi[k][b:b + 1, :])
                old = mem_ref[b, pl.ds(idx, 1), :]
                new = (1.0 - g_s) * old + g_s * wrow
                mem_ref[b, pl.ds(idx, 1), :] = new
                mem_bf[b, pl.ds(idx, 1), :] = _rbf16(new)
        for k in range(WK):
            lw = jnp.where(iota_m == wtopi[k], float(t), lw)
        cat = jnp.concatenate([s0n, s1n], axis=-1)
        mu = jnp.mean(cat, axis=-1, keepdims=True)
        var = jnp.mean((cat - mu) * (cat - mu), axis=-1, keepdims=True)
        normed = (cat - mu) / jnp.sqrt(var + 1e-5) * gam[...] + bet[...]
        out_ref[:, t, :] = _mm(normed, wh[...]) + bh[...]
        s0, s1 = s0n, s1n


def _sc_embed_gather(emb, ids_flat):
    info = plsc.get_sparse_core_info()
    nw = info.num_cores * info.num_subcores
    bt = B * T
    b_per_w = bt // nw
    mesh = plsc.VectorSubcoreMesh(core_axis_name="c", subcore_axis_name="s")

    @functools.partial(
        pl.kernel, mesh=mesh,
        out_type=jax.ShapeDtypeStruct((bt, D), jnp.float32),
        scratch_types=[
            pltpu.VMEM((b_per_w,), jnp.int32),
            pltpu.VMEM((b_per_w, D), jnp.float32),
            pltpu.SemaphoreType.DMA,
        ],
    )
    def gather(table_hbm, idx_hbm, out_hbm, idx_v, rows_v, sem):
        wid = lax.axis_index("s") * info.num_cores + lax.axis_index("c")
        base = wid * b_per_w
        pltpu.sync_copy(idx_hbm.at[pl.ds(base, b_per_w)], idx_v)
        pltpu.async_copy(table_hbm.at[idx_v], rows_v, sem).wait()
        pltpu.sync_copy(rows_v, out_hbm.at[pl.ds(base, b_per_w)])

    return gather(emb, ids_flat)


def kernel(input_ids, emb, Wq, Wk, Wv, Wr, Wt0, bt0, Wt1, bt1, Ww, bw,
           Wg, bg, Whead, bhead, gamma, beta, memory0, local_buffer0,
           read_ema0, last_write0):
    del local_buffer0
    ids_flat = input_ids.reshape(-1).astype(jnp.int32)
    e_all = _sc_embed_gather(emb, ids_flat).reshape(B, T, D)

    grid = (B // BB,)
    w_spec2 = lambda a: pl.BlockSpec(a.shape, lambda i: (0, 0))

    def row1(x):
        return x.reshape(1, -1)

    out = pl.pallas_call(
        _body,
        grid=grid,
        in_specs=[
            pl.BlockSpec((BB, T, D), lambda i: (i, 0, 0)),
            pl.BlockSpec((BB, M, DM), lambda i: (i, 0, 0)),
            pl.BlockSpec((BB, M), lambda i: (i, 0)),
            pl.BlockSpec((BB, M), lambda i: (i, 0)),
            w_spec2(Wq), w_spec2(Wk), w_spec2(Wv), w_spec2(Wr),
            w_spec2(Wt0), w_spec2(bt0.reshape(1, -1)),
            w_spec2(Wt1), w_spec2(bt1.reshape(1, -1)),
            w_spec2(Ww), w_spec2(bw.reshape(1, -1)),
            w_spec2(Wg), w_spec2(bg.reshape(1, -1)),
            w_spec2(Whead), w_spec2(bhead.reshape(1, -1)),
            w_spec2(gamma.reshape(1, -1)), w_spec2(beta.reshape(1, -1)),
        ],
        out_specs=pl.BlockSpec((BB, T, OUT), lambda i: (i, 0, 0)),
        out_shape=jax.ShapeDtypeStruct((B, T, OUT), jnp.float32),
        scratch_shapes=[pltpu.VMEM((BB, M, DM), jnp.float32)],
        compiler_params=pltpu.CompilerParams(
            dimension_semantics=("arbitrary",),
        ),
    )(e_all, memory0, read_ema0, last_write0,
      Wq, Wk, Wv, Wr, Wt0, row1(bt0), Wt1, row1(bt1), Ww, row1(bw),
      Wg, row1(bg), Whead, row1(bhead), row1(gamma), row1(beta))
    return out
